# Initial kernel scaffold; baseline (speedup 1.0000x reference)
import jax
import jax.numpy as jnp
from jax.experimental import pallas as pl


def kernel(x, edge_index, node_pairs, W1, b1, W2, b2, Wfc, bfc):
    raise NotImplementedError("write your pallas kernel here")


# speedup vs baseline: 23.3749x; 23.3749x over previous
import functools

import jax
import jax.numpy as jnp
from jax import lax
from jax.experimental import pallas as pl
from jax.experimental.pallas import tpu as pltpu
from jax.experimental.pallas import tpu_sc as plsc

NC = 2
NS = 16
NW = NC * NS
LANES = 16


def _make_deg_kernel(N, E):
    assert E % NW == 0
    epw = E // NW
    K = 125
    nchunk = epw // K
    assert nchunk * K == epw
    rpt = N // NS
    assert rpt * NS == N and rpt % 8 == 0

    mesh = plsc.VectorSubcoreMesh(core_axis_name="c", subcore_axis_name="s")

    @functools.partial(
        pl.kernel,
        out_type=jax.ShapeDtypeStruct((NC, N, 8), jnp.float32),
        mesh=mesh,
        scratch_types=[
            pltpu.VMEM_SHARED((N, 8), jnp.float32),
            pltpu.VMEM((nchunk, K), jnp.int32),
            pltpu.VMEM((K, 8), jnp.float32),
            pltpu.VMEM((rpt, 8), jnp.float32),
        ],
        compiler_params=pltpu.CompilerParams(use_tc_tiling_on_sc=False),
    )
    def deg_kernel(dst_r, ones_r, zeros_r, out_r, acc, idx_v, ones_v, zbuf):
        c = lax.axis_index("c")
        s = lax.axis_index("s")
        w = s * NC + c
        row0 = s * rpt
        pltpu.sync_copy(zeros_r, zbuf)
        pltpu.sync_copy(zbuf, acc.at[pl.ds(row0, rpt), :])
        pltpu.sync_copy(ones_r, ones_v)
        pltpu.sync_copy(dst_r.at[w], idx_v)
        plsc.subcore_barrier()

        def step(j, carry):
            pltpu.sync_copy(ones_v, acc.at[idx_v.at[j]], add=True)
            return carry

        lax.fori_loop(0, nchunk, step, 0)
        plsc.subcore_barrier()
        pltpu.sync_copy(acc.at[pl.ds(row0, rpt), :], zbuf)
        pltpu.sync_copy(zbuf, out_r.at[c, pl.ds(row0, rpt), :])

    return deg_kernel


def _make_agg_kernel(N, D, E):
    epw = E // NW
    K = 125
    nchunk = epw // K
    assert nchunk * K == epw and nchunk % 2 == 0
    rpt = N // NS
    J = 128
    njc = rpt // J
    assert njc * J == rpt

    IB = 8
    nblk = nchunk // IB
    assert nblk * IB == nchunk

    mesh = plsc.VectorSubcoreMesh(core_axis_name="c", subcore_axis_name="s")

    @functools.partial(
        pl.kernel,
        out_type=jax.ShapeDtypeStruct((NC, N, D), jnp.float32),
        mesh=mesh,
        scratch_types=[
            pltpu.VMEM_SHARED((N, D), jnp.float32),
            pltpu.VMEM((IB, K), jnp.int32),
            pltpu.VMEM((IB, K), jnp.int32),
            pltpu.VMEM((J, D), jnp.float32),
            pltpu.VMEM((J, D), jnp.float32),
            pltpu.SemaphoreType.DMA,
            pltpu.SemaphoreType.DMA,
        ],
    )
    def agg_kernel(hp, src_r, dst_r, out_r, accs, srcv, dstv,
                   rows0, rows1, sem0, sem1):
        c = lax.axis_index("c")
        s = lax.axis_index("s")
        w = s * NC + c
        row0 = s * rpt
        for j in range(njc):
            pltpu.sync_copy(hp.at[pl.ds(row0 + j * J, J), :], rows0)
            pltpu.sync_copy(rows0, accs.at[pl.ds(row0 + j * J, J), :])
        plsc.subcore_barrier()

        r0 = rows0.at[pl.ds(0, K), :]
        r1 = rows1.at[pl.ds(0, K), :]

        def blk(b, carry):
            pltpu.sync_copy(src_r.at[w, pl.ds(b * IB, IB), :], srcv)
            pltpu.sync_copy(dst_r.at[w, pl.ds(b * IB, IB), :], dstv)

            def step(i, carry2):
                j0 = i * 2
                d0 = pltpu.async_copy(hp.at[srcv.at[j0]], r0, sem0)
                d1 = pltpu.async_copy(hp.at[srcv.at[j0 + 1]], r1, sem1)
                d0.wait()
                pltpu.sync_copy(r0, accs.at[dstv.at[j0]], add=True)
                d1.wait()
                pltpu.sync_copy(r1, accs.at[dstv.at[j0 + 1]], add=True)
                return carry2

            lax.fori_loop(0, IB // 2, step, 0)
            return carry

        lax.fori_loop(0, nblk, blk, 0)
        plsc.subcore_barrier()
        for j in range(njc):
            pltpu.sync_copy(accs.at[pl.ds(row0 + j * J, J), :], rows0)
            pltpu.sync_copy(rows0, out_r.at[c, pl.ds(row0 + j * J, J), :])

    return agg_kernel


def _make_pair_kernel(N, ppw):
    nv = ppw // LANES
    assert nv * LANES == ppw and ppw % 8 == 0

    mesh = plsc.VectorSubcoreMesh(core_axis_name="c", subcore_axis_name="s")

    @functools.partial(
        pl.kernel,
        out_type=jax.ShapeDtypeStruct((NW * ppw,), jnp.float32),
        mesh=mesh,
        scratch_types=[
            pltpu.VMEM((2 * N,), jnp.float32),
            pltpu.VMEM((ppw,), jnp.int32),
            pltpu.VMEM((ppw,), jnp.int32),
            pltpu.VMEM((ppw,), jnp.float32),
            pltpu.VMEM((LANES,), jnp.float32),
        ],
        compiler_params=pltpu.CompilerParams(needs_layout_passes=False),
    )
    def pair_kernel(uv, pa, pb, bfc_r, out_r, uvv, pav, pbv, outv, bv):
        c = lax.axis_index("c")
        s = lax.axis_index("s")
        w = s * NC + c
        pltpu.sync_copy(uv, uvv)
        pltpu.sync_copy(pa.at[pl.ds(w * ppw, ppw)], pav)
        pltpu.sync_copy(pb.at[pl.ds(w * ppw, ppw)], pbv)
        pltpu.sync_copy(bfc_r, bv)
        bias = bv[...]

        def step(i, carry):
            ia = pav[pl.ds(i * LANES, LANES)]
            ib = pbv[pl.ds(i * LANES, LANES)]
            zu = plsc.load_gather(uvv, [ia * 2])
            zv = plsc.load_gather(uvv, [ib * 2 + 1])
            outv[pl.ds(i * LANES, LANES)] = zu + zv + bias
            return carry

        lax.fori_loop(0, nv, step, 0)
        pltpu.sync_copy(outv, out_r.at[pl.ds(w * ppw, ppw)])

    return pair_kernel


def _mm_scale_body(x_ref, w_ref, dp_ref, out_ref):
    deg = jnp.sum(dp_ref[...], axis=(0, 2)) * 0.125 + 1.0
    dinv = lax.rsqrt(deg)
    g = jnp.dot(x_ref[...], w_ref[...], preferred_element_type=jnp.float32)
    out_ref[...] = g * dinv[:, None]


def _combine_body(acc_ref, hp_ref, dp_ref, b_ref, w_ref, out_ref, *, scale_out):
    deg = jnp.sum(dp_ref[...], axis=(0, 2)) * 0.125 + 1.0
    dinv = lax.rsqrt(deg)
    ssum = acc_ref[0] + acc_ref[1] - hp_ref[...]
    o = jnp.maximum(ssum * dinv[:, None] + b_ref[...], 0.0)
    g = jnp.dot(o, w_ref[...], preferred_element_type=jnp.float32)
    if scale_out:
        g = g * dinv[:, None]
    out_ref[...] = g


def _mm_scale(x, W, dp, bm):
    N, D = x.shape
    grid = N // bm
    return pl.pallas_call(
        _mm_scale_body,
        grid=(grid,),
        in_specs=[
            pl.BlockSpec((bm, D), lambda i: (i, 0)),
            pl.BlockSpec((D, D), lambda i: (0, 0)),
            pl.BlockSpec((NC, bm, 8), lambda i: (0, i, 0)),
        ],
        out_specs=pl.BlockSpec((bm, D), lambda i: (i, 0)),
        out_shape=jax.ShapeDtypeStruct((N, D), jnp.float32),
    )(x, W, dp)


def _combine_mm(acc, hp, dp, b, W, bm, scale_out):
    N, D = hp.shape
    Do = W.shape[1]
    grid = N // bm
    return pl.pallas_call(
        functools.partial(_combine_body, scale_out=scale_out),
        grid=(grid,),
        in_specs=[
            pl.BlockSpec((NC, bm, D), lambda i: (0, i, 0)),
            pl.BlockSpec((bm, D), lambda i: (i, 0)),
            pl.BlockSpec((NC, bm, 8), lambda i: (0, i, 0)),
            pl.BlockSpec((1, D), lambda i: (0, 0)),
            pl.BlockSpec((D, Do), lambda i: (0, 0)),
        ],
        out_specs=pl.BlockSpec((bm, Do), lambda i: (i, 0)),
        out_shape=jax.ShapeDtypeStruct((N, Do), jnp.float32),
    )(acc, hp, dp, b, W)


def kernel(x, edge_index, node_pairs, W1, b1, W2, b2, Wfc, bfc):
    N, D = x.shape
    E = edge_index.shape[1]
    P = node_pairs.shape[0]
    K = 125
    nchunk = E // (NW * K)
    BM = 2048
    NPAD = -(-N // BM) * BM
    assert (NPAD // NS) % 128 == 0

    x_p = jnp.pad(x, ((0, NPAD - N), (0, 0)))
    src_r = edge_index[0].astype(jnp.int32).reshape(NW, nchunk, K)
    dst_r = edge_index[1].astype(jnp.int32).reshape(NW, nchunk, K)
    ones8 = jnp.ones((K, 8), jnp.float32)
    zeros8 = jnp.zeros((NPAD // NS, 8), jnp.float32)

    ppw = -(-P // NW)
    ppw = -(-ppw // LANES) * LANES
    pad = NW * ppw - P
    pa = jnp.pad(node_pairs[:, 0].astype(jnp.int32), (0, pad))
    pb = jnp.pad(node_pairs[:, 1].astype(jnp.int32), (0, pad))
    bfc16 = jnp.broadcast_to(bfc.astype(jnp.float32), (LANES,))
    Wuv = jnp.stack([Wfc[:D, 0], Wfc[D:, 0]], axis=1)

    deg_parts = _make_deg_kernel(NPAD, E)(dst_r, ones8, zeros8)
    h1p = _mm_scale(x_p, W1, deg_parts, BM)
    agg = _make_agg_kernel(NPAD, D, E)
    acc1 = agg(h1p, src_r, dst_r)
    h2p = _combine_mm(acc1, h1p, deg_parts, b1.reshape(1, D), W2, BM, True)
    acc2 = agg(h2p, src_r, dst_r)
    uv = _combine_mm(acc2, h2p, deg_parts, b2.reshape(1, D), Wuv, BM, False)
    outp = _make_pair_kernel(NPAD, ppw)(uv.reshape(-1), pa, pb, bfc16)
    return outp[:P].reshape(P, 1)


# speedup vs baseline: 28.4195x; 1.2158x over previous
import functools

import jax
import jax.numpy as jnp
from jax import lax
from jax.experimental import pallas as pl
from jax.experimental.pallas import tpu as pltpu
from jax.experimental.pallas import tpu_sc as plsc

NC = 2
NS = 16
NW = NC * NS
LANES = 16


def _make_deg_kernel(N, E):
    assert E % NW == 0
    epw = E // NW
    K = 125
    nchunk = epw // K
    assert nchunk * K == epw
    rpt = N // NS
    assert rpt * NS == N and rpt % 8 == 0

    mesh = plsc.VectorSubcoreMesh(core_axis_name="c", subcore_axis_name="s")

    @functools.partial(
        pl.kernel,
        out_type=jax.ShapeDtypeStruct((NC, N, 8), jnp.float32),
        mesh=mesh,
        scratch_types=[
            pltpu.VMEM_SHARED((N, 8), jnp.float32),
            pltpu.VMEM((nchunk, K), jnp.int32),
            pltpu.VMEM((K, 8), jnp.float32),
            pltpu.VMEM((rpt, 8), jnp.float32),
        ],
        compiler_params=pltpu.CompilerParams(use_tc_tiling_on_sc=False),
    )
    def deg_kernel(dst_r, ones_r, zeros_r, out_r, acc, idx_v, ones_v, zbuf):
        c = lax.axis_index("c")
        s = lax.axis_index("s")
        w = s * NC + c
        row0 = s * rpt
        pltpu.sync_copy(zeros_r, zbuf)
        pltpu.sync_copy(zbuf, acc.at[pl.ds(row0, rpt), :])
        pltpu.sync_copy(ones_r, ones_v)
        pltpu.sync_copy(dst_r.at[w], idx_v)
        plsc.subcore_barrier()

        def step(j, carry):
            pltpu.sync_copy(ones_v, acc.at[idx_v.at[j]], add=True)
            return carry

        lax.fori_loop(0, nchunk, step, 0)
        plsc.subcore_barrier()
        pltpu.sync_copy(acc.at[pl.ds(row0, rpt), :], zbuf)
        pltpu.sync_copy(zbuf, out_r.at[c, pl.ds(row0, rpt), :])

    return deg_kernel


def _make_agg_kernel(N, D, E, NOUT):
    epw = E // NW
    K = 125
    nchunk = epw // K
    assert nchunk * K == epw and nchunk % 2 == 0
    rpt = N // NS
    J = K
    njc = rpt // J
    assert njc * J == rpt

    IB = 8
    nblk = nchunk // IB
    assert nblk * IB == nchunk

    mesh = plsc.VectorSubcoreMesh(core_axis_name="c", subcore_axis_name="s")

    @functools.partial(
        pl.kernel,
        out_type=jax.ShapeDtypeStruct((NC, NOUT, D), jnp.float32),
        mesh=mesh,
        scratch_types=[
            pltpu.VMEM_SHARED((N, D), jnp.float32),
            pltpu.VMEM((nchunk, K), jnp.int32),
            pltpu.VMEM((IB, K), jnp.int32),
            pltpu.VMEM((K, D), jnp.float32),
            pltpu.VMEM((K, D), jnp.float32),
            pltpu.SemaphoreType.DMA,
            pltpu.SemaphoreType.DMA,
            pltpu.SemaphoreType.DMA,
            pltpu.SemaphoreType.DMA,
        ],
        compiler_params=pltpu.CompilerParams(use_tc_tiling_on_sc=False),
    )
    def agg_kernel(hp, src_r, dst_r, out_r, accs, srcv, dstv,
                   r0, r1, semg0, semg1, sems0, sems1):
        c = lax.axis_index("c")
        s = lax.axis_index("s")
        w = s * NC + c
        row0 = s * rpt
        pltpu.sync_copy(src_r.at[w], srcv)
        for j in range(njc):
            pltpu.sync_copy(hp.at[pl.ds(row0 + j * J, J), :], r0)
            pltpu.sync_copy(r0, accs.at[pl.ds(row0 + j * J, J), :])
        plsc.subcore_barrier()

        pltpu.async_copy(hp.at[srcv.at[0]], r0, semg0)

        def blk(b, carry):
            pltpu.sync_copy(dst_r.at[w, pl.ds(b * IB, IB), :], dstv)

            def step(i, carry2):
                j0 = b * IB + i * 2
                d1 = pltpu.async_copy(hp.at[srcv.at[j0 + 1]], r1, semg1)
                pltpu.make_async_copy(hp.at[srcv.at[j0]], r0, semg0).wait()
                s0 = pltpu.async_copy(r0, accs.at[dstv.at[i * 2]], sems0,
                                      add=True)
                d1.wait()
                s1 = pltpu.async_copy(r1, accs.at[dstv.at[i * 2 + 1]], sems1,
                                      add=True)
                s0.wait()
                nxt = jnp.minimum(j0 + 2, nchunk - 1)
                pltpu.async_copy(hp.at[srcv.at[nxt]], r0, semg0)
                s1.wait()
                return carry2

            lax.fori_loop(0, IB // 2, step, 0)
            return carry

        lax.fori_loop(0, nblk, blk, 0)
        pltpu.make_async_copy(hp.at[srcv.at[nchunk - 1]], r0, semg0).wait()
        plsc.subcore_barrier()
        for j in range(njc):
            pltpu.sync_copy(accs.at[pl.ds(row0 + j * J, J), :], r0)
            pltpu.sync_copy(r0, out_r.at[c, pl.ds(row0 + j * J, J), :])

    return agg_kernel


def _make_pair_kernel(N, ppw):
    nv = ppw // LANES
    assert nv * LANES == ppw and ppw % 8 == 0

    mesh = plsc.VectorSubcoreMesh(core_axis_name="c", subcore_axis_name="s")

    @functools.partial(
        pl.kernel,
        out_type=jax.ShapeDtypeStruct((NW * ppw,), jnp.float32),
        mesh=mesh,
        scratch_types=[
            pltpu.VMEM((2 * N,), jnp.float32),
            pltpu.VMEM((ppw,), jnp.int32),
            pltpu.VMEM((ppw,), jnp.int32),
            pltpu.VMEM((ppw,), jnp.float32),
            pltpu.VMEM((LANES,), jnp.float32),
        ],
        compiler_params=pltpu.CompilerParams(needs_layout_passes=False),
    )
    def pair_kernel(uv, pa, pb, bfc_r, out_r, uvv, pav, pbv, outv, bv):
        c = lax.axis_index("c")
        s = lax.axis_index("s")
        w = s * NC + c
        pltpu.sync_copy(uv, uvv)
        pltpu.sync_copy(pa.at[pl.ds(w * ppw, ppw)], pav)
        pltpu.sync_copy(pb.at[pl.ds(w * ppw, ppw)], pbv)
        pltpu.sync_copy(bfc_r, bv)
        bias = bv[...]

        def step(i, carry):
            ia = pav[pl.ds(i * LANES, LANES)]
            ib = pbv[pl.ds(i * LANES, LANES)]
            zu = plsc.load_gather(uvv, [ia * 2])
            zv = plsc.load_gather(uvv, [ib * 2 + 1])
            outv[pl.ds(i * LANES, LANES)] = zu + zv + bias
            return carry

        lax.fori_loop(0, nv, step, 0)
        pltpu.sync_copy(outv, out_r.at[pl.ds(w * ppw, ppw)])

    return pair_kernel


def _mm_scale_body(x_ref, w_ref, dp_ref, out_ref):
    deg = jnp.sum(dp_ref[...], axis=(0, 2)) * 0.125 + 1.0
    dinv = lax.rsqrt(deg)
    g = jnp.dot(x_ref[...], w_ref[...], preferred_element_type=jnp.float32)
    out_ref[...] = g * dinv[:, None]


def _combine_body(acc_ref, hp_ref, dp_ref, b_ref, w_ref, out_ref, *, scale_out):
    deg = jnp.sum(dp_ref[...], axis=(0, 2)) * 0.125 + 1.0
    dinv = lax.rsqrt(deg)
    ssum = acc_ref[0] + acc_ref[1] - hp_ref[...]
    o = jnp.maximum(ssum * dinv[:, None] + b_ref[...], 0.0)
    g = jnp.dot(o, w_ref[...], preferred_element_type=jnp.float32)
    if scale_out:
        g = g * dinv[:, None]
    out_ref[...] = g


def _mm_scale(x, W, dp, bm):
    N, D = x.shape
    grid = N // bm
    return pl.pallas_call(
        _mm_scale_body,
        grid=(grid,),
        in_specs=[
            pl.BlockSpec((bm, D), lambda i: (i, 0)),
            pl.BlockSpec((D, D), lambda i: (0, 0)),
            pl.BlockSpec((NC, bm, 8), lambda i: (0, i, 0)),
        ],
        out_specs=pl.BlockSpec((bm, D), lambda i: (i, 0)),
        out_shape=jax.ShapeDtypeStruct((N, D), jnp.float32),
    )(x, W, dp)


def _combine_mm(acc, hp, dp, b, W, bm, scale_out):
    N, D = hp.shape
    Do = W.shape[1]
    grid = N // bm
    return pl.pallas_call(
        functools.partial(_combine_body, scale_out=scale_out),
        grid=(grid,),
        in_specs=[
            pl.BlockSpec((NC, bm, D), lambda i: (0, i, 0)),
            pl.BlockSpec((bm, D), lambda i: (i, 0)),
            pl.BlockSpec((NC, bm, 8), lambda i: (0, i, 0)),
            pl.BlockSpec((1, D), lambda i: (0, 0)),
            pl.BlockSpec((D, Do), lambda i: (0, 0)),
        ],
        out_specs=pl.BlockSpec((bm, Do), lambda i: (i, 0)),
        out_shape=jax.ShapeDtypeStruct((N, Do), jnp.float32),
    )(acc, hp, dp, b, W)


def kernel(x, edge_index, node_pairs, W1, b1, W2, b2, Wfc, bfc):
    N, D = x.shape
    E = edge_index.shape[1]
    P = node_pairs.shape[0]
    K = 125
    nchunk = E // (NW * K)
    BM = 2048
    NPAD = -(-N // BM) * BM
    assert (NPAD // NS) % 128 == 0

    x_p = jnp.pad(x, ((0, NPAD - N), (0, 0)))
    src_r = edge_index[0].astype(jnp.int32).reshape(NW, nchunk, K)
    dst_r = edge_index[1].astype(jnp.int32).reshape(NW, nchunk, K)
    ones8 = jnp.ones((K, 8), jnp.float32)
    zeros8 = jnp.zeros((NPAD // NS, 8), jnp.float32)

    ppw = -(-P // NW)
    ppw = -(-ppw // LANES) * LANES
    pad = NW * ppw - P
    pa = jnp.pad(node_pairs[:, 0].astype(jnp.int32), (0, pad))
    pb = jnp.pad(node_pairs[:, 1].astype(jnp.int32), (0, pad))
    bfc16 = jnp.broadcast_to(bfc.astype(jnp.float32), (LANES,))
    Wuv = jnp.stack([Wfc[:D, 0], Wfc[D:, 0]], axis=1)

    deg_parts = _make_deg_kernel(NPAD, E)(dst_r, ones8, zeros8)
    h1p = _mm_scale(x_p, W1, deg_parts, BM)
    agg = _make_agg_kernel(N, D, E, NPAD)
    acc1 = agg(h1p, src_r, dst_r)
    h2p = _combine_mm(acc1, h1p, deg_parts, b1.reshape(1, D), W2, BM, True)
    acc2 = agg(h2p, src_r, dst_r)
    uv = _combine_mm(acc2, h2p, deg_parts, b2.reshape(1, D), Wuv, BM, False)
    outp = _make_pair_kernel(NPAD, ppw)(uv.reshape(-1), pa, pb, bfc16)
    return outp[:P].reshape(P, 1)


# speedup vs baseline: 28.4649x; 1.0016x over previous
import functools

import jax
import jax.numpy as jnp
from jax import lax
from jax.experimental import pallas as pl
from jax.experimental.pallas import tpu as pltpu
from jax.experimental.pallas import tpu_sc as plsc

NC = 2
NS = 16
NW = NC * NS
LANES = 16


def _make_deg_kernel(N, E):
    assert E % NW == 0
    epw = E // NW
    K = 125
    nchunk = epw // K
    assert nchunk * K == epw
    rpt = N // NS
    assert rpt * NS == N and rpt % 8 == 0

    mesh = plsc.VectorSubcoreMesh(core_axis_name="c", subcore_axis_name="s")

    @functools.partial(
        pl.kernel,
        out_type=jax.ShapeDtypeStruct((NC, N, 8), jnp.float32),
        mesh=mesh,
        scratch_types=[
            pltpu.VMEM_SHARED((N, 8), jnp.float32),
            pltpu.VMEM((nchunk, K), jnp.int32),
            pltpu.VMEM((K, 8), jnp.float32),
            pltpu.VMEM((rpt, 8), jnp.float32),
        ],
        compiler_params=pltpu.CompilerParams(use_tc_tiling_on_sc=False),
    )
    def deg_kernel(dst_r, ones_r, zeros_r, out_r, acc, idx_v, ones_v, zbuf):
        c = lax.axis_index("c")
        s = lax.axis_index("s")
        w = s * NC + c
        row0 = s * rpt
        pltpu.sync_copy(zeros_r, zbuf)
        pltpu.sync_copy(zbuf, acc.at[pl.ds(row0, rpt), :])
        pltpu.sync_copy(ones_r, ones_v)
        pltpu.sync_copy(dst_r.at[w], idx_v)
        plsc.subcore_barrier()

        def step(j, carry):
            pltpu.sync_copy(ones_v, acc.at[idx_v.at[j]], add=True)
            return carry

        lax.fori_loop(0, nchunk, step, 0)
        plsc.subcore_barrier()
        pltpu.sync_copy(acc.at[pl.ds(row0, rpt), :], zbuf)
        pltpu.sync_copy(zbuf, out_r.at[c, pl.ds(row0, rpt), :])

    return deg_kernel


def _make_agg_kernel(N, D, E, NOUT):
    DH = D // 2
    ept = E // NS
    K = 125
    nchunk = ept // K
    assert nchunk * K == ept and nchunk % 4 == 0
    rpt = N // NS
    J = K
    njc = rpt // J
    assert njc * J == rpt

    mesh = plsc.VectorSubcoreMesh(core_axis_name="c", subcore_axis_name="s")

    @functools.partial(
        pl.kernel,
        out_type=jax.ShapeDtypeStruct((NC, NOUT, DH), jnp.float32),
        mesh=mesh,
        scratch_types=[
            pltpu.VMEM_SHARED((N, DH), jnp.float32),
            pltpu.VMEM((nchunk, K), jnp.int32),
            pltpu.VMEM((nchunk, K), jnp.int32),
            pltpu.VMEM((K, DH), jnp.float32),
            pltpu.VMEM((K, DH), jnp.float32),
            pltpu.VMEM((K, DH), jnp.float32),
            pltpu.VMEM((K, DH), jnp.float32),
            pltpu.SemaphoreType.DMA,
            pltpu.SemaphoreType.DMA,
            pltpu.SemaphoreType.DMA,
            pltpu.SemaphoreType.DMA,
            pltpu.SemaphoreType.DMA,
            pltpu.SemaphoreType.DMA,
            pltpu.SemaphoreType.DMA,
            pltpu.SemaphoreType.DMA,
        ],
        compiler_params=pltpu.CompilerParams(use_tc_tiling_on_sc=False),
    )
    def agg_kernel(hp, src_r, dst_r, out_r, accs, srcv, dstv,
                   rA, rB, rC, rD, gA, gB, gC, gD, sA, sB, sC, sD):
        c = lax.axis_index("c")
        s = lax.axis_index("s")
        row0 = s * rpt
        hpc = hp.at[c]
        pltpu.sync_copy(src_r.at[s], srcv)
        pltpu.sync_copy(dst_r.at[s], dstv)
        for j in range(njc):
            pltpu.sync_copy(hpc.at[pl.ds(row0 + j * J, J), :], rA)
            pltpu.sync_copy(rA, accs.at[pl.ds(row0 + j * J, J), :])
        plsc.subcore_barrier()

        pltpu.async_copy(hpc.at[srcv.at[0]], rA, gA)
        pltpu.async_copy(hpc.at[srcv.at[1]], rB, gB)
        pltpu.async_copy(hpc.at[srcv.at[2]], rC, gC)

        def wait_g(j, r, sem):
            pltpu.make_async_copy(hpc.at[srcv.at[j]], r, sem).wait()

        def step(i, carry):
            j0 = i * 4
            last = nchunk - 1
            pltpu.async_copy(hpc.at[srcv.at[j0 + 3]], rD, gD)
            wait_g(j0, rA, gA)
            dsA = pltpu.async_copy(rA, accs.at[dstv.at[j0]], sA, add=True)
            wait_g(j0 + 1, rB, gB)
            dsB = pltpu.async_copy(rB, accs.at[dstv.at[j0 + 1]], sB, add=True)
            dsA.wait()
            pltpu.async_copy(hpc.at[srcv.at[jnp.minimum(j0 + 4, last)]], rA, gA)
            wait_g(j0 + 2, rC, gC)
            dsC = pltpu.async_copy(rC, accs.at[dstv.at[j0 + 2]], sC, add=True)
            dsB.wait()
            pltpu.async_copy(hpc.at[srcv.at[jnp.minimum(j0 + 5, last)]], rB, gB)
            wait_g(j0 + 3, rD, gD)
            dsD = pltpu.async_copy(rD, accs.at[dstv.at[j0 + 3]], sD, add=True)
            dsC.wait()
            pltpu.async_copy(hpc.at[srcv.at[jnp.minimum(j0 + 6, last)]], rC, gC)
            dsD.wait()
            return carry

        lax.fori_loop(0, nchunk // 4, step, 0)
        wait_g(nchunk - 1, rA, gA)
        wait_g(nchunk - 1, rB, gB)
        wait_g(nchunk - 1, rC, gC)
        plsc.subcore_barrier()
        for j in range(njc):
            pltpu.sync_copy(accs.at[pl.ds(row0 + j * J, J), :], rA)
            pltpu.sync_copy(rA, out_r.at[c, pl.ds(row0 + j * J, J), :])

    return agg_kernel


def _make_pair_kernel(N, ppw):
    nv = ppw // LANES
    assert nv * LANES == ppw and ppw % 8 == 0

    mesh = plsc.VectorSubcoreMesh(core_axis_name="c", subcore_axis_name="s")

    @functools.partial(
        pl.kernel,
        out_type=jax.ShapeDtypeStruct((NW * ppw,), jnp.float32),
        mesh=mesh,
        scratch_types=[
            pltpu.VMEM((2 * N,), jnp.float32),
            pltpu.VMEM((ppw,), jnp.int32),
            pltpu.VMEM((ppw,), jnp.int32),
            pltpu.VMEM((ppw,), jnp.float32),
            pltpu.VMEM((LANES,), jnp.float32),
        ],
        compiler_params=pltpu.CompilerParams(needs_layout_passes=False),
    )
    def pair_kernel(uv, pa, pb, bfc_r, out_r, uvv, pav, pbv, outv, bv):
        c = lax.axis_index("c")
        s = lax.axis_index("s")
        w = s * NC + c
        pltpu.sync_copy(uv, uvv)
        pltpu.sync_copy(pa.at[pl.ds(w * ppw, ppw)], pav)
        pltpu.sync_copy(pb.at[pl.ds(w * ppw, ppw)], pbv)
        pltpu.sync_copy(bfc_r, bv)
        bias = bv[...]

        def step(i, carry):
            ia = pav[pl.ds(i * LANES, LANES)]
            ib = pbv[pl.ds(i * LANES, LANES)]
            zu = plsc.load_gather(uvv, [ia * 2])
            zv = plsc.load_gather(uvv, [ib * 2 + 1])
            outv[pl.ds(i * LANES, LANES)] = zu + zv + bias
            return carry

        lax.fori_loop(0, nv, step, 0)
        pltpu.sync_copy(outv, out_r.at[pl.ds(w * ppw, ppw)])

    return pair_kernel


def _dinv_of(dp_ref):
    deg = jnp.sum(dp_ref[...], axis=(0, 2)) * 0.125 + 1.0
    return lax.rsqrt(deg)


def _mm_scale_body(x_ref, w_ref, dp_ref, out_ref):
    dinv = _dinv_of(dp_ref)
    g = jnp.dot(x_ref[...], w_ref[...], preferred_element_type=jnp.float32)
    g = g * dinv[:, None]
    DH = g.shape[1] // 2
    out_ref[0] = g[:, :DH]
    out_ref[1] = g[:, DH:]


def _combine_body(acc_ref, dp_ref, b_ref, w_ref, out_ref, *, split_out):
    dinv = _dinv_of(dp_ref)
    pre = jnp.concatenate([acc_ref[0], acc_ref[1]], axis=1)
    o = jnp.maximum(pre * dinv[:, None] + b_ref[...], 0.0)
    g = jnp.dot(o, w_ref[...], preferred_element_type=jnp.float32)
    if split_out:
        g = g * dinv[:, None]
        DH = g.shape[1] // 2
        out_ref[0] = g[:, :DH]
        out_ref[1] = g[:, DH:]
    else:
        out_ref[...] = g


def _mm_scale(x, W, dp, bm):
    N, D = x.shape
    grid = N // bm
    return pl.pallas_call(
        _mm_scale_body,
        grid=(grid,),
        in_specs=[
            pl.BlockSpec((bm, D), lambda i: (i, 0)),
            pl.BlockSpec((D, D), lambda i: (0, 0)),
            pl.BlockSpec((NC, bm, 8), lambda i: (0, i, 0)),
        ],
        out_specs=pl.BlockSpec((NC, bm, D // 2), lambda i: (0, i, 0)),
        out_shape=jax.ShapeDtypeStruct((NC, N, D // 2), jnp.float32),
    )(x, W, dp)


def _combine_mm(acc, dp, b, W, bm, split_out):
    _, N, DH = acc.shape
    D = 2 * DH
    Do = W.shape[1]
    grid = N // bm
    if split_out:
        out_spec = pl.BlockSpec((NC, bm, Do // 2), lambda i: (0, i, 0))
        out_shape = jax.ShapeDtypeStruct((NC, N, Do // 2), jnp.float32)
    else:
        out_spec = pl.BlockSpec((bm, Do), lambda i: (i, 0))
        out_shape = jax.ShapeDtypeStruct((N, Do), jnp.float32)
    return pl.pallas_call(
        functools.partial(_combine_body, split_out=split_out),
        grid=(grid,),
        in_specs=[
            pl.BlockSpec((NC, bm, DH), lambda i: (0, i, 0)),
            pl.BlockSpec((NC, bm, 8), lambda i: (0, i, 0)),
            pl.BlockSpec((1, D), lambda i: (0, 0)),
            pl.BlockSpec((D, Do), lambda i: (0, 0)),
        ],
        out_specs=out_spec,
        out_shape=out_shape,
    )(acc, dp, b, W)


def kernel(x, edge_index, node_pairs, W1, b1, W2, b2, Wfc, bfc):
    N, D = x.shape
    E = edge_index.shape[1]
    P = node_pairs.shape[0]
    K = 125
    nchunk = E // (NW * K)
    BM = 2048
    NPAD = -(-N // BM) * BM
    assert (NPAD // NS) % 128 == 0

    x_p = jnp.pad(x, ((0, NPAD - N), (0, 0)))
    src = edge_index[0].astype(jnp.int32)
    dst = edge_index[1].astype(jnp.int32)
    dst_r = dst.reshape(NW, nchunk, K)
    src_agg = src.reshape(NS, E // (NS * K), K)
    dst_agg = dst.reshape(NS, E // (NS * K), K)
    ones8 = jnp.ones((K, 8), jnp.float32)
    zeros8 = jnp.zeros((NPAD // NS, 8), jnp.float32)

    ppw = -(-P // NW)
    ppw = -(-ppw // LANES) * LANES
    pad = NW * ppw - P
    pa = jnp.pad(node_pairs[:, 0].astype(jnp.int32), (0, pad))
    pb = jnp.pad(node_pairs[:, 1].astype(jnp.int32), (0, pad))
    bfc16 = jnp.broadcast_to(bfc.astype(jnp.float32), (LANES,))
    Wuv = jnp.stack([Wfc[:D, 0], Wfc[D:, 0]], axis=1)

    deg_parts = _make_deg_kernel(NPAD, E)(dst_r, ones8, zeros8)
    h1p = _mm_scale(x_p, W1, deg_parts, BM)
    agg = _make_agg_kernel(N, D, E, NPAD)
    acc1 = agg(h1p, src_agg, dst_agg)
    h2p = _combine_mm(acc1, deg_parts, b1.reshape(1, D), W2, BM, True)
    acc2 = agg(h2p, src_agg, dst_agg)
    uv = _combine_mm(acc2, deg_parts, b2.reshape(1, D), Wuv, BM, False)
    outp = _make_pair_kernel(NPAD, ppw)(uv.reshape(-1), pa, pb, bfc16)
    return outp[:P].reshape(P, 1)


# speedup vs baseline: 32.5463x; 1.1434x over previous
import functools

import jax
import jax.numpy as jnp
from jax import lax
from jax.experimental import pallas as pl
from jax.experimental.pallas import tpu as pltpu
from jax.experimental.pallas import tpu_sc as plsc

NC = 2
NS = 16
NW = NC * NS
LANES = 16


def _make_deg_kernel(N, E):
    assert E % NW == 0
    epw = E // NW
    K = 125
    nchunk = epw // K
    assert nchunk * K == epw
    rpt = N // NS
    assert rpt * NS == N and rpt % 8 == 0

    mesh = plsc.VectorSubcoreMesh(core_axis_name="c", subcore_axis_name="s")

    @functools.partial(
        pl.kernel,
        out_type=jax.ShapeDtypeStruct((NC, N, 8), jnp.float32),
        mesh=mesh,
        scratch_types=[
            pltpu.VMEM_SHARED((N, 8), jnp.float32),
            pltpu.VMEM((nchunk, K), jnp.int32),
            pltpu.VMEM((K, 8), jnp.float32),
            pltpu.VMEM((rpt, 8), jnp.float32),
        ],
        compiler_params=pltpu.CompilerParams(use_tc_tiling_on_sc=False),
    )
    def deg_kernel(dst_r, ones_r, zeros_r, out_r, acc, idx_v, ones_v, zbuf):
        c = lax.axis_index("c")
        s = lax.axis_index("s")
        w = s * NC + c
        row0 = s * rpt
        pltpu.sync_copy(zeros_r, zbuf)
        pltpu.sync_copy(zbuf, acc.at[pl.ds(row0, rpt), :])
        pltpu.sync_copy(ones_r, ones_v)
        pltpu.sync_copy(dst_r.at[w], idx_v)
        plsc.subcore_barrier()

        def step(j, carry):
            pltpu.sync_copy(ones_v, acc.at[idx_v.at[j]], add=True)
            return carry

        lax.fori_loop(0, nchunk, step, 0)
        plsc.subcore_barrier()
        pltpu.sync_copy(acc.at[pl.ds(row0, rpt), :], zbuf)
        pltpu.sync_copy(zbuf, out_r.at[c, pl.ds(row0, rpt), :])

    return deg_kernel


def _make_agg_kernel(N, D, E, NOUT):
    DH = D // 2
    ept = E // NS
    K = 125
    nchunk = ept // K
    assert nchunk * K == ept and nchunk % 4 == 0
    rpt = N // NS
    J = K
    njc = rpt // J
    assert njc * J == rpt

    mesh = plsc.VectorSubcoreMesh(core_axis_name="c", subcore_axis_name="s")

    @functools.partial(
        pl.kernel,
        out_type=jax.ShapeDtypeStruct((2 * NOUT, DH), jnp.float32),
        mesh=mesh,
        scratch_types=[
            pltpu.VMEM_SHARED((N, DH), jnp.float32),
            pltpu.VMEM((nchunk, K), jnp.int32),
            pltpu.VMEM((nchunk, K), jnp.int32),
            pltpu.VMEM((njc, K), jnp.int32),
            pltpu.VMEM((K, DH), jnp.float32),
            pltpu.VMEM((K, DH), jnp.float32),
            pltpu.VMEM((K, DH), jnp.float32),
            pltpu.VMEM((K, DH), jnp.float32),
            pltpu.SemaphoreType.DMA,
            pltpu.SemaphoreType.DMA,
            pltpu.SemaphoreType.DMA,
            pltpu.SemaphoreType.DMA,
            pltpu.SemaphoreType.DMA,
            pltpu.SemaphoreType.DMA,
            pltpu.SemaphoreType.DMA,
            pltpu.SemaphoreType.DMA,
        ],
        compiler_params=pltpu.CompilerParams(use_tc_tiling_on_sc=False),
    )
    def agg_kernel(hp, srcA, srcB, dst_r, oiA, oiB, out_r, accs, srcv, dstv,
                   oiv, rA, rB, rC, rD, gA, gB, gC, gD, sA, sB, sC, sD):
        c = lax.axis_index("c")
        s = lax.axis_index("s")
        row0 = s * rpt

        @pl.when(c == 0)
        def _():
            pltpu.sync_copy(srcA.at[s], srcv)
            pltpu.sync_copy(oiA.at[s], oiv)

        @pl.when(c == 1)
        def _():
            pltpu.sync_copy(srcB.at[s], srcv)
            pltpu.sync_copy(oiB.at[s], oiv)

        pltpu.sync_copy(dst_r.at[s], dstv)
        for j in range(njc):
            pltpu.async_copy(hp.at[oiv.at[j]], rA, gA).wait()
            pltpu.sync_copy(rA, accs.at[pl.ds(row0 + j * J, J), :])
        plsc.subcore_barrier()

        pltpu.async_copy(hp.at[srcv.at[0]], rA, gA)
        pltpu.async_copy(hp.at[srcv.at[1]], rB, gB)
        pltpu.async_copy(hp.at[srcv.at[2]], rC, gC)

        def wait_g(j, r, sem):
            pltpu.make_async_copy(hp.at[srcv.at[j]], r, sem).wait()

        def step(i, carry):
            j0 = i * 4
            last = nchunk - 1
            pltpu.async_copy(hp.at[srcv.at[j0 + 3]], rD, gD)
            wait_g(j0, rA, gA)
            dsA = pltpu.async_copy(rA, accs.at[dstv.at[j0]], sA, add=True)
            wait_g(j0 + 1, rB, gB)
            dsB = pltpu.async_copy(rB, accs.at[dstv.at[j0 + 1]], sB, add=True)
            dsA.wait()
            pltpu.async_copy(hp.at[srcv.at[jnp.minimum(j0 + 4, last)]], rA, gA)
            wait_g(j0 + 2, rC, gC)
            dsC = pltpu.async_copy(rC, accs.at[dstv.at[j0 + 2]], sC, add=True)
            dsB.wait()
            pltpu.async_copy(hp.at[srcv.at[jnp.minimum(j0 + 5, last)]], rB, gB)
            wait_g(j0 + 3, rD, gD)
            dsD = pltpu.async_copy(rD, accs.at[dstv.at[j0 + 3]], sD, add=True)
            dsC.wait()
            pltpu.async_copy(hp.at[srcv.at[jnp.minimum(j0 + 6, last)]], rC, gC)
            dsD.wait()
            return carry

        lax.fori_loop(0, nchunk // 4, step, 0)
        wait_g(nchunk - 1, rA, gA)
        wait_g(nchunk - 1, rB, gB)
        wait_g(nchunk - 1, rC, gC)
        plsc.subcore_barrier()
        for j in range(njc):
            pltpu.sync_copy(accs.at[pl.ds(row0 + j * J, J), :], rA)
            pltpu.async_copy(rA, out_r.at[oiv.at[j]], gA).wait()

    return agg_kernel


def _make_pair_kernel(N, ppw):
    nv = ppw // LANES
    assert nv * LANES == ppw and ppw % 8 == 0

    mesh = plsc.VectorSubcoreMesh(core_axis_name="c", subcore_axis_name="s")

    @functools.partial(
        pl.kernel,
        out_type=jax.ShapeDtypeStruct((NW * ppw,), jnp.float32),
        mesh=mesh,
        scratch_types=[
            pltpu.VMEM((2 * N,), jnp.float32),
            pltpu.VMEM((ppw,), jnp.int32),
            pltpu.VMEM((ppw,), jnp.int32),
            pltpu.VMEM((ppw,), jnp.float32),
            pltpu.VMEM((LANES,), jnp.float32),
        ],
        compiler_params=pltpu.CompilerParams(needs_layout_passes=False),
    )
    def pair_kernel(uv, pa, pb, bfc_r, out_r, uvv, pav, pbv, outv, bv):
        c = lax.axis_index("c")
        s = lax.axis_index("s")
        w = s * NC + c
        pltpu.sync_copy(uv, uvv)
        pltpu.sync_copy(pa.at[pl.ds(w * ppw, ppw)], pav)
        pltpu.sync_copy(pb.at[pl.ds(w * ppw, ppw)], pbv)
        pltpu.sync_copy(bfc_r, bv)
        bias = bv[...]

        def step(i, carry):
            ia = pav[pl.ds(i * LANES, LANES)]
            ib = pbv[pl.ds(i * LANES, LANES)]
            zu = plsc.load_gather(uvv, [ia * 2])
            zv = plsc.load_gather(uvv, [ib * 2 + 1])
            outv[pl.ds(i * LANES, LANES)] = zu + zv + bias
            return carry

        lax.fori_loop(0, nv, step, 0)
        pltpu.sync_copy(outv, out_r.at[pl.ds(w * ppw, ppw)])

    return pair_kernel


def _dinv_of(dp_ref):
    deg = jnp.sum(dp_ref[...], axis=(0, 2)) * 0.125 + 1.0
    return lax.rsqrt(deg)


def _mm_scale_body(x_ref, w_ref, dp_ref, out_ref):
    dinv = _dinv_of(dp_ref)
    g = jnp.dot(x_ref[...], w_ref[...], preferred_element_type=jnp.float32)
    out_ref[...] = g * dinv[:, None]


def _combine_body(acc_ref, dp_ref, b_ref, w_ref, out_ref, *, scale_out):
    dinv = _dinv_of(dp_ref)
    o = jnp.maximum(acc_ref[...] * dinv[:, None] + b_ref[...], 0.0)
    g = jnp.dot(o, w_ref[...], preferred_element_type=jnp.float32)
    if scale_out:
        g = g * dinv[:, None]
    out_ref[...] = g


def _mm_scale(x, W, dp, bm):
    N, D = x.shape
    grid = N // bm
    return pl.pallas_call(
        _mm_scale_body,
        grid=(grid,),
        in_specs=[
            pl.BlockSpec((bm, D), lambda i: (i, 0)),
            pl.BlockSpec((D, D), lambda i: (0, 0)),
            pl.BlockSpec((NC, bm, 8), lambda i: (0, i, 0)),
        ],
        out_specs=pl.BlockSpec((bm, D), lambda i: (i, 0)),
        out_shape=jax.ShapeDtypeStruct((N, D), jnp.float32),
    )(x, W, dp)


def _combine_mm(acc, dp, b, W, bm, scale_out):
    N, D = acc.shape
    Do = W.shape[1]
    grid = N // bm
    return pl.pallas_call(
        functools.partial(_combine_body, scale_out=scale_out),
        grid=(grid,),
        in_specs=[
            pl.BlockSpec((bm, D), lambda i: (i, 0)),
            pl.BlockSpec((NC, bm, 8), lambda i: (0, i, 0)),
            pl.BlockSpec((1, D), lambda i: (0, 0)),
            pl.BlockSpec((D, Do), lambda i: (0, 0)),
        ],
        out_specs=pl.BlockSpec((bm, Do), lambda i: (i, 0)),
        out_shape=jax.ShapeDtypeStruct((N, Do), jnp.float32),
    )(acc, dp, b, W)


def kernel(x, edge_index, node_pairs, W1, b1, W2, b2, Wfc, bfc):
    N, D = x.shape
    E = edge_index.shape[1]
    P = node_pairs.shape[0]
    K = 125
    nchunk = E // (NW * K)
    BM = 2048
    NPAD = -(-N // BM) * BM
    assert (NPAD // NS) % 128 == 0

    x_p = jnp.pad(x, ((0, NPAD - N), (0, 0)))
    src = edge_index[0].astype(jnp.int32)
    dst = edge_index[1].astype(jnp.int32)
    dst_r = dst.reshape(NW, nchunk, K)
    nch_agg = E // (NS * K)
    srcA = (src * 2).reshape(NS, nch_agg, K)
    srcB = (src * 2 + 1).reshape(NS, nch_agg, K)
    dst_agg = dst.reshape(NS, nch_agg, K)
    nid = jnp.arange(N, dtype=jnp.int32)
    oiA = (nid * 2).reshape(NS, (N // NS) // K, K)
    oiB = (nid * 2 + 1).reshape(NS, (N // NS) // K, K)
    ones8 = jnp.ones((K, 8), jnp.float32)
    zeros8 = jnp.zeros((NPAD // NS, 8), jnp.float32)

    ppw = -(-P // NW)
    ppw = -(-ppw // LANES) * LANES
    pad = NW * ppw - P
    pa = jnp.pad(node_pairs[:, 0].astype(jnp.int32), (0, pad))
    pb = jnp.pad(node_pairs[:, 1].astype(jnp.int32), (0, pad))
    bfc16 = jnp.broadcast_to(bfc.astype(jnp.float32), (LANES,))
    Wuv = jnp.stack([Wfc[:D, 0], Wfc[D:, 0]], axis=1)

    deg_parts = _make_deg_kernel(NPAD, E)(dst_r, ones8, zeros8)
    h1p = _mm_scale(x_p, W1, deg_parts, BM)
    agg = _make_agg_kernel(N, D, E, NPAD)
    acc1 = agg(h1p.reshape(2 * NPAD, D // 2), srcA, srcB, dst_agg, oiA, oiB)
    h2p = _combine_mm(acc1.reshape(NPAD, D), deg_parts,
                      b1.reshape(1, D), W2, BM, True)
    acc2 = agg(h2p.reshape(2 * NPAD, D // 2), srcA, srcB, dst_agg, oiA, oiB)
    uv = _combine_mm(acc2.reshape(NPAD, D), deg_parts,
                     b2.reshape(1, D), Wuv, BM, False)
    outp = _make_pair_kernel(NPAD, ppw)(uv.reshape(-1), pa, pb, bfc16)
    return outp[:P].reshape(P, 1)


# speedup vs baseline: 33.5523x; 1.0309x over previous
import functools

import jax
import jax.numpy as jnp
from jax import lax
from jax.experimental import pallas as pl
from jax.experimental.pallas import tpu as pltpu
from jax.experimental.pallas import tpu_sc as plsc

NC = 2
NS = 16
NW = NC * NS
LANES = 16


def _make_deg_kernel(N, E):
    assert E % NW == 0
    epw = E // NW
    K = 125
    nchunk = epw // K
    assert nchunk * K == epw
    rpt = N // NS
    assert rpt * NS == N

    mesh = plsc.VectorSubcoreMesh(core_axis_name="c", subcore_axis_name="s")

    @functools.partial(
        pl.kernel,
        out_type=jax.ShapeDtypeStruct((NC, N, 8), jnp.float32),
        mesh=mesh,
        scratch_types=[
            pltpu.VMEM_SHARED((N, 8), jnp.float32),
            pltpu.VMEM((nchunk, K), jnp.int32),
            pltpu.VMEM((K, 8), jnp.float32),
            pltpu.VMEM((rpt, 8), jnp.float32),
        ],
        compiler_params=pltpu.CompilerParams(use_tc_tiling_on_sc=False),
    )
    def deg_kernel(dst_r, ones_r, zeros_r, out_r, acc, idx_v, ones_v, zbuf):
        c = lax.axis_index("c")
        s = lax.axis_index("s")
        w = s * NC + c
        row0 = s * rpt
        pltpu.sync_copy(zeros_r, zbuf)
        pltpu.sync_copy(zbuf, acc.at[pl.ds(row0, rpt), :])
        pltpu.sync_copy(ones_r, ones_v)
        pltpu.sync_copy(dst_r.at[w], idx_v)
        plsc.subcore_barrier()

        def step(j, carry):
            pltpu.sync_copy(ones_v, acc.at[idx_v.at[j]], add=True)
            return carry

        lax.fori_loop(0, nchunk, step, 0)
        plsc.subcore_barrier()
        pltpu.sync_copy(acc.at[pl.ds(row0, rpt), :], zbuf)
        pltpu.sync_copy(zbuf, out_r.at[c, pl.ds(row0, rpt), :])

    return deg_kernel


def _make_agg_kernel(N, D, E, NOUT):
    DH = D // 2
    ept = E // NS
    K = 125
    nchunk = ept // K
    assert nchunk * K == ept and nchunk % 4 == 0
    rpt = N // NS
    J = K
    njc = rpt // J
    assert njc * J == rpt

    mesh = plsc.VectorSubcoreMesh(core_axis_name="c", subcore_axis_name="s")

    @functools.partial(
        pl.kernel,
        out_type=jax.ShapeDtypeStruct((2 * NOUT, DH), jnp.float32),
        mesh=mesh,
        scratch_types=[
            pltpu.VMEM_SHARED((N, DH), jnp.float32),
            pltpu.VMEM((nchunk, K), jnp.int32),
            pltpu.VMEM((nchunk, K), jnp.int32),
            pltpu.VMEM((njc, K), jnp.int32),
            pltpu.VMEM((K, DH), jnp.float32),
            pltpu.VMEM((K, DH), jnp.float32),
            pltpu.VMEM((K, DH), jnp.float32),
            pltpu.VMEM((K, DH), jnp.float32),
            pltpu.SemaphoreType.DMA,
            pltpu.SemaphoreType.DMA,
            pltpu.SemaphoreType.DMA,
            pltpu.SemaphoreType.DMA,
            pltpu.SemaphoreType.DMA,
            pltpu.SemaphoreType.DMA,
            pltpu.SemaphoreType.DMA,
            pltpu.SemaphoreType.DMA,
        ],
        compiler_params=pltpu.CompilerParams(use_tc_tiling_on_sc=False),
    )
    def agg_kernel(hp, srcA, srcB, dst_r, oiA, oiB, out_r, accs, srcv, dstv,
                   oiv, rA, rB, rC, rD, gA, gB, gC, gD, sA, sB, sC, sD):
        c = lax.axis_index("c")
        s = lax.axis_index("s")
        row0 = s * rpt

        @pl.when(c == 0)
        def _():
            pltpu.sync_copy(srcA.at[s], srcv)
            pltpu.sync_copy(oiA.at[s], oiv)

        @pl.when(c == 1)
        def _():
            pltpu.sync_copy(srcB.at[s], srcv)
            pltpu.sync_copy(oiB.at[s], oiv)

        pltpu.sync_copy(dst_r.at[s], dstv)
        bufs = [rA, rB, rC, rD]
        gsem = [gA, gB, gC, gD]
        for j in range(min(njc, 4)):
            pltpu.async_copy(hp.at[oiv.at[j]], bufs[j % 4], gsem[j % 4])
        for j in range(njc):
            pltpu.make_async_copy(hp.at[oiv.at[j]], bufs[j % 4],
                                  gsem[j % 4]).wait()
            pltpu.sync_copy(bufs[j % 4], accs.at[pl.ds(row0 + j * J, J), :])
            if j + 4 < njc:
                pltpu.async_copy(hp.at[oiv.at[j + 4]], bufs[j % 4],
                                 gsem[j % 4])
        plsc.subcore_barrier()

        pltpu.async_copy(hp.at[srcv.at[0]], rA, gA)
        pltpu.async_copy(hp.at[srcv.at[1]], rB, gB)
        pltpu.async_copy(hp.at[srcv.at[2]], rC, gC)

        def wait_g(j, r, sem):
            pltpu.make_async_copy(hp.at[srcv.at[j]], r, sem).wait()

        def step(i, carry):
            j0 = i * 4
            last = nchunk - 1
            pltpu.async_copy(hp.at[srcv.at[j0 + 3]], rD, gD)
            wait_g(j0, rA, gA)
            dsA = pltpu.async_copy(rA, accs.at[dstv.at[j0]], sA, add=True)
            wait_g(j0 + 1, rB, gB)
            dsB = pltpu.async_copy(rB, accs.at[dstv.at[j0 + 1]], sB, add=True)
            dsA.wait()
            pltpu.async_copy(hp.at[srcv.at[jnp.minimum(j0 + 4, last)]], rA, gA)
            wait_g(j0 + 2, rC, gC)
            dsC = pltpu.async_copy(rC, accs.at[dstv.at[j0 + 2]], sC, add=True)
            dsB.wait()
            pltpu.async_copy(hp.at[srcv.at[jnp.minimum(j0 + 5, last)]], rB, gB)
            wait_g(j0 + 3, rD, gD)
            dsD = pltpu.async_copy(rD, accs.at[dstv.at[j0 + 3]], sD, add=True)
            dsC.wait()
            pltpu.async_copy(hp.at[srcv.at[jnp.minimum(j0 + 6, last)]], rC, gC)
            dsD.wait()
            return carry

        lax.fori_loop(0, nchunk // 4, step, 0)
        wait_g(nchunk - 1, rA, gA)
        wait_g(nchunk - 1, rB, gB)
        wait_g(nchunk - 1, rC, gC)
        plsc.subcore_barrier()
        for j in range(njc):
            if j >= 4:
                pltpu.make_async_copy(bufs[j % 4], out_r.at[oiv.at[j - 4]],
                                      gsem[j % 4]).wait()
            pltpu.sync_copy(accs.at[pl.ds(row0 + j * J, J), :], bufs[j % 4])
            pltpu.async_copy(bufs[j % 4], out_r.at[oiv.at[j]], gsem[j % 4])
        for j in range(max(njc - 4, 0), njc):
            pltpu.make_async_copy(bufs[j % 4], out_r.at[oiv.at[j]],
                                  gsem[j % 4]).wait()

    return agg_kernel


def _make_pair_kernel(N, ppw):
    nv = ppw // LANES
    assert nv * LANES == ppw and ppw % 8 == 0

    mesh = plsc.VectorSubcoreMesh(core_axis_name="c", subcore_axis_name="s")

    @functools.partial(
        pl.kernel,
        out_type=jax.ShapeDtypeStruct((NW * ppw,), jnp.float32),
        mesh=mesh,
        scratch_types=[
            pltpu.VMEM((2 * N,), jnp.float32),
            pltpu.VMEM((ppw,), jnp.int32),
            pltpu.VMEM((ppw,), jnp.int32),
            pltpu.VMEM((ppw,), jnp.float32),
            pltpu.VMEM((LANES,), jnp.float32),
        ],
        compiler_params=pltpu.CompilerParams(needs_layout_passes=False),
    )
    def pair_kernel(uv, pa, pb, bfc_r, out_r, uvv, pav, pbv, outv, bv):
        c = lax.axis_index("c")
        s = lax.axis_index("s")
        w = s * NC + c
        pltpu.sync_copy(uv, uvv)
        pltpu.sync_copy(pa.at[pl.ds(w * ppw, ppw)], pav)
        pltpu.sync_copy(pb.at[pl.ds(w * ppw, ppw)], pbv)
        pltpu.sync_copy(bfc_r, bv)
        bias = bv[...]

        def step(i, carry):
            ia = pav[pl.ds(i * LANES, LANES)]
            ib = pbv[pl.ds(i * LANES, LANES)]
            zu = plsc.load_gather(uvv, [ia * 2])
            zv = plsc.load_gather(uvv, [ib * 2 + 1])
            outv[pl.ds(i * LANES, LANES)] = zu + zv + bias
            return carry

        lax.fori_loop(0, nv, step, 0)
        pltpu.sync_copy(outv, out_r.at[pl.ds(w * ppw, ppw)])

    return pair_kernel


def _dinv_of(dp_ref):
    deg = jnp.sum(dp_ref[...], axis=(0, 2)) * 0.125 + 1.0
    return lax.rsqrt(deg)


def _mm_scale_body(x_ref, w_ref, dp_ref, out_ref):
    dinv = _dinv_of(dp_ref)
    g = jnp.dot(x_ref[...], w_ref[...], preferred_element_type=jnp.float32)
    out_ref[...] = g * dinv[:, None]


def _combine_body(acc_ref, dp_ref, b_ref, w_ref, out_ref, *, scale_out):
    dinv = _dinv_of(dp_ref)
    o = jnp.maximum(acc_ref[...] * dinv[:, None] + b_ref[...], 0.0)
    g = jnp.dot(o, w_ref[...], preferred_element_type=jnp.float32)
    if scale_out:
        g = g * dinv[:, None]
    out_ref[...] = g


def _mm_scale(x, W, dp, bm):
    N, D = x.shape
    grid = N // bm
    return pl.pallas_call(
        _mm_scale_body,
        grid=(grid,),
        in_specs=[
            pl.BlockSpec((bm, D), lambda i: (i, 0)),
            pl.BlockSpec((D, D), lambda i: (0, 0)),
            pl.BlockSpec((NC, bm, 8), lambda i: (0, i, 0)),
        ],
        out_specs=pl.BlockSpec((bm, D), lambda i: (i, 0)),
        out_shape=jax.ShapeDtypeStruct((N, D), jnp.float32),
    )(x, W, dp)


def _combine_mm(acc, dp, b, W, bm, scale_out):
    N, D = acc.shape
    Do = W.shape[1]
    grid = N // bm
    return pl.pallas_call(
        functools.partial(_combine_body, scale_out=scale_out),
        grid=(grid,),
        in_specs=[
            pl.BlockSpec((bm, D), lambda i: (i, 0)),
            pl.BlockSpec((NC, bm, 8), lambda i: (0, i, 0)),
            pl.BlockSpec((1, D), lambda i: (0, 0)),
            pl.BlockSpec((D, Do), lambda i: (0, 0)),
        ],
        out_specs=pl.BlockSpec((bm, Do), lambda i: (i, 0)),
        out_shape=jax.ShapeDtypeStruct((N, Do), jnp.float32),
    )(acc, dp, b, W)


def kernel(x, edge_index, node_pairs, W1, b1, W2, b2, Wfc, bfc):
    N, D = x.shape
    E = edge_index.shape[1]
    P = node_pairs.shape[0]
    K = 125
    nchunk = E // (NW * K)
    BM = 2000
    assert N % BM == 0

    src = edge_index[0].astype(jnp.int32)
    dst = edge_index[1].astype(jnp.int32)
    dst_r = dst.reshape(NW, nchunk, K)
    nch_agg = E // (NS * K)
    srcA = (src * 2).reshape(NS, nch_agg, K)
    srcB = (src * 2 + 1).reshape(NS, nch_agg, K)
    dst_agg = dst.reshape(NS, nch_agg, K)
    nid = jnp.arange(N, dtype=jnp.int32)
    oiA = (nid * 2).reshape(NS, (N // NS) // K, K)
    oiB = (nid * 2 + 1).reshape(NS, (N // NS) // K, K)
    ones8 = jnp.ones((K, 8), jnp.float32)
    zeros8 = jnp.zeros((N // NS, 8), jnp.float32)

    ppw = -(-P // NW)
    ppw = -(-ppw // LANES) * LANES
    pad = NW * ppw - P
    pa = jnp.pad(node_pairs[:, 0].astype(jnp.int32), (0, pad))
    pb = jnp.pad(node_pairs[:, 1].astype(jnp.int32), (0, pad))
    bfc16 = jnp.broadcast_to(bfc.astype(jnp.float32), (LANES,))
    Wuv = jnp.stack([Wfc[:D, 0], Wfc[D:, 0]], axis=1)

    deg_parts = _make_deg_kernel(N, E)(dst_r, ones8, zeros8)
    h1p = _mm_scale(x, W1, deg_parts, BM)
    agg = _make_agg_kernel(N, D, E, N)
    acc1 = agg(h1p.reshape(2 * N, D // 2), srcA, srcB, dst_agg, oiA, oiB)
    h2p = _combine_mm(acc1.reshape(N, D), deg_parts,
                      b1.reshape(1, D), W2, BM, True)
    acc2 = agg(h2p.reshape(2 * N, D // 2), srcA, srcB, dst_agg, oiA, oiB)
    uv = _combine_mm(acc2.reshape(N, D), deg_parts,
                     b2.reshape(1, D), Wuv, BM, False)
    outp = _make_pair_kernel(N, ppw)(uv.reshape(-1), pa, pb, bfc16)
    return outp[:P].reshape(P, 1)


# speedup vs baseline: 34.0672x; 1.0153x over previous
import functools

import jax
import jax.numpy as jnp
from jax import lax
from jax.experimental import pallas as pl
from jax.experimental.pallas import tpu as pltpu
from jax.experimental.pallas import tpu_sc as plsc

NC = 2
NS = 16
NW = NC * NS
LANES = 16


def _make_deg_kernel(N, E):
    assert E % NW == 0
    epw = E // NW
    K = 125
    nchunk = epw // K
    assert nchunk * K == epw
    rpt = N // NS
    assert rpt * NS == N

    mesh = plsc.VectorSubcoreMesh(core_axis_name="c", subcore_axis_name="s")

    @functools.partial(
        pl.kernel,
        out_type=jax.ShapeDtypeStruct((NC, N, 8), jnp.float32),
        mesh=mesh,
        scratch_types=[
            pltpu.VMEM_SHARED((N, 8), jnp.float32),
            pltpu.VMEM((nchunk, K), jnp.int32),
            pltpu.VMEM((K, 8), jnp.float32),
            pltpu.VMEM((rpt, 8), jnp.float32),
            pltpu.SemaphoreType.DMA,
            pltpu.SemaphoreType.DMA,
            pltpu.SemaphoreType.DMA,
            pltpu.SemaphoreType.DMA,
        ],
        compiler_params=pltpu.CompilerParams(use_tc_tiling_on_sc=False),
    )
    def deg_kernel(dst_r, ones_r, zeros_r, out_r, acc, idx_v, ones_v, zbuf,
                   s0, s1, s2, s3):
        c = lax.axis_index("c")
        s = lax.axis_index("s")
        w = s * NC + c
        row0 = s * rpt
        pltpu.sync_copy(zeros_r, zbuf)
        pltpu.sync_copy(zbuf, acc.at[pl.ds(row0, rpt), :])
        pltpu.sync_copy(ones_r, ones_v)
        pltpu.sync_copy(dst_r.at[w], idx_v)
        plsc.subcore_barrier()

        sems = [s0, s1, s2, s3]
        assert nchunk % 4 == 0

        def prime(k):
            pltpu.async_copy(ones_v, acc.at[idx_v.at[k]], sems[k], add=True)

        for k in range(4):
            prime(k)

        def step2(i, carry):
            j0 = (i + 1) * 4
            for k in range(4):
                pltpu.make_async_copy(
                    ones_v, acc.at[idx_v.at[j0 + k - 4]], sems[k]).wait()
                pltpu.async_copy(ones_v, acc.at[idx_v.at[j0 + k]], sems[k],
                                 add=True)
            return carry

        lax.fori_loop(0, nchunk // 4 - 1, step2, 0)
        for k in range(4):
            pltpu.make_async_copy(
                ones_v, acc.at[idx_v.at[nchunk - 4 + k]], sems[k]).wait()
        plsc.subcore_barrier()
        pltpu.sync_copy(acc.at[pl.ds(row0, rpt), :], zbuf)
        pltpu.sync_copy(zbuf, out_r.at[c, pl.ds(row0, rpt), :])

    return deg_kernel


def _make_agg_kernel(N, D, E, NOUT):
    DH = D // 2
    ept = E // NS
    K = 125
    nchunk = ept // K
    assert nchunk * K == ept and nchunk % 4 == 0
    rpt = N // NS
    J = K
    njc = rpt // J
    assert njc * J == rpt

    mesh = plsc.VectorSubcoreMesh(core_axis_name="c", subcore_axis_name="s")

    @functools.partial(
        pl.kernel,
        out_type=jax.ShapeDtypeStruct((2 * NOUT, DH), jnp.float32),
        mesh=mesh,
        scratch_types=[
            pltpu.VMEM_SHARED((N, DH), jnp.float32),
            pltpu.VMEM((nchunk, K), jnp.int32),
            pltpu.VMEM((nchunk, K), jnp.int32),
            pltpu.VMEM((njc, K), jnp.int32),
            pltpu.VMEM((K, DH), jnp.float32),
            pltpu.VMEM((K, DH), jnp.float32),
            pltpu.VMEM((K, DH), jnp.float32),
            pltpu.VMEM((K, DH), jnp.float32),
            pltpu.SemaphoreType.DMA,
            pltpu.SemaphoreType.DMA,
            pltpu.SemaphoreType.DMA,
            pltpu.SemaphoreType.DMA,
            pltpu.SemaphoreType.DMA,
            pltpu.SemaphoreType.DMA,
            pltpu.SemaphoreType.DMA,
            pltpu.SemaphoreType.DMA,
        ],
        compiler_params=pltpu.CompilerParams(use_tc_tiling_on_sc=False),
    )
    def agg_kernel(hp, srcA, srcB, dst_r, oiA, oiB, out_r, accs, srcv, dstv,
                   oiv, rA, rB, rC, rD, gA, gB, gC, gD, sA, sB, sC, sD):
        c = lax.axis_index("c")
        s = lax.axis_index("s")
        row0 = s * rpt

        @pl.when(c == 0)
        def _():
            pltpu.sync_copy(srcA.at[s], srcv)
            pltpu.sync_copy(oiA.at[s], oiv)

        @pl.when(c == 1)
        def _():
            pltpu.sync_copy(srcB.at[s], srcv)
            pltpu.sync_copy(oiB.at[s], oiv)

        pltpu.sync_copy(dst_r.at[s], dstv)
        bufs = [rA, rB, rC, rD]
        gsem = [gA, gB, gC, gD]
        for j in range(min(njc, 4)):
            pltpu.async_copy(hp.at[oiv.at[j]], bufs[j % 4], gsem[j % 4])
        for j in range(njc):
            pltpu.make_async_copy(hp.at[oiv.at[j]], bufs[j % 4],
                                  gsem[j % 4]).wait()
            pltpu.sync_copy(bufs[j % 4], accs.at[pl.ds(row0 + j * J, J), :])
            if j + 4 < njc:
                pltpu.async_copy(hp.at[oiv.at[j + 4]], bufs[j % 4],
                                 gsem[j % 4])
        plsc.subcore_barrier()

        pltpu.async_copy(hp.at[srcv.at[0]], rA, gA)
        pltpu.async_copy(hp.at[srcv.at[1]], rB, gB)
        pltpu.async_copy(hp.at[srcv.at[2]], rC, gC)

        def wait_g(j, r, sem):
            pltpu.make_async_copy(hp.at[srcv.at[j]], r, sem).wait()

        def step(i, carry):
            j0 = i * 4
            last = nchunk - 1
            pltpu.async_copy(hp.at[srcv.at[j0 + 3]], rD, gD)
            wait_g(j0, rA, gA)
            dsA = pltpu.async_copy(rA, accs.at[dstv.at[j0]], sA, add=True)
            wait_g(j0 + 1, rB, gB)
            dsB = pltpu.async_copy(rB, accs.at[dstv.at[j0 + 1]], sB, add=True)
            dsA.wait()
            pltpu.async_copy(hp.at[srcv.at[jnp.minimum(j0 + 4, last)]], rA, gA)
            wait_g(j0 + 2, rC, gC)
            dsC = pltpu.async_copy(rC, accs.at[dstv.at[j0 + 2]], sC, add=True)
            dsB.wait()
            pltpu.async_copy(hp.at[srcv.at[jnp.minimum(j0 + 5, last)]], rB, gB)
            wait_g(j0 + 3, rD, gD)
            dsD = pltpu.async_copy(rD, accs.at[dstv.at[j0 + 3]], sD, add=True)
            dsC.wait()
            pltpu.async_copy(hp.at[srcv.at[jnp.minimum(j0 + 6, last)]], rC, gC)
            dsD.wait()
            return carry

        lax.fori_loop(0, nchunk // 4, step, 0)
        wait_g(nchunk - 1, rA, gA)
        wait_g(nchunk - 1, rB, gB)
        wait_g(nchunk - 1, rC, gC)
        plsc.subcore_barrier()
        for j in range(njc):
            if j >= 4:
                pltpu.make_async_copy(bufs[j % 4], out_r.at[oiv.at[j - 4]],
                                      gsem[j % 4]).wait()
            pltpu.sync_copy(accs.at[pl.ds(row0 + j * J, J), :], bufs[j % 4])
            pltpu.async_copy(bufs[j % 4], out_r.at[oiv.at[j]], gsem[j % 4])
        for j in range(max(njc - 4, 0), njc):
            pltpu.make_async_copy(bufs[j % 4], out_r.at[oiv.at[j]],
                                  gsem[j % 4]).wait()

    return agg_kernel


def _make_pair_kernel(N, ppw):
    nv = ppw // LANES
    assert nv * LANES == ppw and ppw % 8 == 0

    mesh = plsc.VectorSubcoreMesh(core_axis_name="c", subcore_axis_name="s")

    @functools.partial(
        pl.kernel,
        out_type=jax.ShapeDtypeStruct((NW * ppw,), jnp.float32),
        mesh=mesh,
        scratch_types=[
            pltpu.VMEM((2 * N,), jnp.float32),
            pltpu.VMEM((ppw,), jnp.int32),
            pltpu.VMEM((ppw,), jnp.int32),
            pltpu.VMEM((ppw,), jnp.float32),
            pltpu.VMEM((LANES,), jnp.float32),
        ],
        compiler_params=pltpu.CompilerParams(needs_layout_passes=False),
    )
    def pair_kernel(uv, pa, pb, bfc_r, out_r, uvv, pav, pbv, outv, bv):
        c = lax.axis_index("c")
        s = lax.axis_index("s")
        w = s * NC + c
        pltpu.sync_copy(uv, uvv)
        pltpu.sync_copy(pa.at[pl.ds(w * ppw, ppw)], pav)
        pltpu.sync_copy(pb.at[pl.ds(w * ppw, ppw)], pbv)
        pltpu.sync_copy(bfc_r, bv)
        bias = bv[...]

        def step(i, carry):
            ia = pav[pl.ds(i * LANES, LANES)]
            ib = pbv[pl.ds(i * LANES, LANES)]
            zu = plsc.load_gather(uvv, [ia * 2])
            zv = plsc.load_gather(uvv, [ib * 2 + 1])
            outv[pl.ds(i * LANES, LANES)] = zu + zv + bias
            return carry

        lax.fori_loop(0, nv, step, 0)
        pltpu.sync_copy(outv, out_r.at[pl.ds(w * ppw, ppw)])

    return pair_kernel


def _dinv_of(dp_ref):
    deg = jnp.sum(dp_ref[...], axis=(0, 2)) * 0.125 + 1.0
    return lax.rsqrt(deg)


def _mm_scale_body(x_ref, w_ref, dp_ref, out_ref):
    dinv = _dinv_of(dp_ref)
    g = jnp.dot(x_ref[...], w_ref[...], preferred_element_type=jnp.float32)
    out_ref[...] = g * dinv[:, None]


def _combine_body(acc_ref, dp_ref, b_ref, w_ref, out_ref, *, scale_out):
    dinv = _dinv_of(dp_ref)
    o = jnp.maximum(acc_ref[...] * dinv[:, None] + b_ref[...], 0.0)
    g = jnp.dot(o, w_ref[...], preferred_element_type=jnp.float32)
    if scale_out:
        g = g * dinv[:, None]
    out_ref[...] = g


def _mm_scale(x, W, dp, bm):
    N, D = x.shape
    grid = N // bm
    return pl.pallas_call(
        _mm_scale_body,
        grid=(grid,),
        in_specs=[
            pl.BlockSpec((bm, D), lambda i: (i, 0)),
            pl.BlockSpec((D, D), lambda i: (0, 0)),
            pl.BlockSpec((NC, bm, 8), lambda i: (0, i, 0)),
        ],
        out_specs=pl.BlockSpec((bm, D), lambda i: (i, 0)),
        out_shape=jax.ShapeDtypeStruct((N, D), jnp.float32),
    )(x, W, dp)


def _combine_mm(acc, dp, b, W, bm, scale_out):
    N, D = acc.shape
    Do = W.shape[1]
    grid = N // bm
    return pl.pallas_call(
        functools.partial(_combine_body, scale_out=scale_out),
        grid=(grid,),
        in_specs=[
            pl.BlockSpec((bm, D), lambda i: (i, 0)),
            pl.BlockSpec((NC, bm, 8), lambda i: (0, i, 0)),
            pl.BlockSpec((1, D), lambda i: (0, 0)),
            pl.BlockSpec((D, Do), lambda i: (0, 0)),
        ],
        out_specs=pl.BlockSpec((bm, Do), lambda i: (i, 0)),
        out_shape=jax.ShapeDtypeStruct((N, Do), jnp.float32),
    )(acc, dp, b, W)


def kernel(x, edge_index, node_pairs, W1, b1, W2, b2, Wfc, bfc):
    N, D = x.shape
    E = edge_index.shape[1]
    P = node_pairs.shape[0]
    K = 125
    nchunk = E // (NW * K)
    BM = 2000
    assert N % BM == 0

    src = edge_index[0].astype(jnp.int32)
    dst = edge_index[1].astype(jnp.int32)
    dst_r = dst.reshape(NW, nchunk, K)
    nch_agg = E // (NS * K)
    srcA = (src * 2).reshape(NS, nch_agg, K)
    srcB = (src * 2 + 1).reshape(NS, nch_agg, K)
    dst_agg = dst.reshape(NS, nch_agg, K)
    nid = jnp.arange(N, dtype=jnp.int32)
    oiA = (nid * 2).reshape(NS, (N // NS) // K, K)
    oiB = (nid * 2 + 1).reshape(NS, (N // NS) // K, K)
    ones8 = jnp.ones((K, 8), jnp.float32)
    zeros8 = jnp.zeros((N // NS, 8), jnp.float32)

    ppw = -(-P // NW)
    ppw = -(-ppw // LANES) * LANES
    pad = NW * ppw - P
    pa = jnp.pad(node_pairs[:, 0].astype(jnp.int32), (0, pad))
    pb = jnp.pad(node_pairs[:, 1].astype(jnp.int32), (0, pad))
    bfc16 = jnp.broadcast_to(bfc.astype(jnp.float32), (LANES,))
    Wuv = jnp.stack([Wfc[:D, 0], Wfc[D:, 0]], axis=1)

    deg_parts = _make_deg_kernel(N, E)(dst_r, ones8, zeros8)
    h1p = _mm_scale(x, W1, deg_parts, BM)
    agg = _make_agg_kernel(N, D, E, N)
    acc1 = agg(h1p.reshape(2 * N, D // 2), srcA, srcB, dst_agg, oiA, oiB)
    h2p = _combine_mm(acc1.reshape(N, D), deg_parts,
                      b1.reshape(1, D), W2, BM, True)
    acc2 = agg(h2p.reshape(2 * N, D // 2), srcA, srcB, dst_agg, oiA, oiB)
    uv = _combine_mm(acc2.reshape(N, D), deg_parts,
                     b2.reshape(1, D), Wuv, BM, False)
    outp = _make_pair_kernel(N, ppw)(uv.reshape(-1), pa, pb, bfc16)
    return outp[:P].reshape(P, 1)
